# baseline (speedup 1.0000x reference)
import dataclasses

import jax
import jax.numpy as jnp
from jax import lax
from jax.experimental import pallas as pl
from jax.experimental.pallas import tpu as pltpu
from jax.experimental.pallas import tpu_sc as plsc

NV = 100000
NC = 100000
NE = 3200000
NG = 256
PCOEF = 3.0

NSUB = 16
NW = 32
EPW = NE // NW
ECH = 5000
NCHUNK = EPW // ECH
VCH = 1000
NVCH = NV // VCH
NBUF = 4


def _sc_body(xv_hbm, adjp_hbm, adjn_hbm, out_hbm,
             cidx0, cidx1, cidx2, cidx3, vidx0, vidx1, vidx2, vidx3,
             xv0, xv1, xv2, xv3, na0, na1, na2, na3, nb0, nb1, nb2, nb3,
             cb, xvs, accn, accd,
             semL, semG, semS):
    cidx = [cidx0, cidx1, cidx2, cidx3]
    vidx = [vidx0, vidx1, vidx2, vidx3]
    xval = [xv0, xv1, xv2, xv3]
    na = [na0, na1, na2, na3]
    nb = [nb0, nb1, nb2, nb3]
    cid = lax.axis_index("c")
    sid = lax.axis_index("s")
    w = cid * NSUB + sid

    for k in range((NVCH + NSUB - 1) // NSUB):
        t = sid + NSUB * k

        @pl.when(t < NVCH)
        def _():
            off = t * VCH
            pltpu.sync_copy(xv_hbm.at[pl.ds(off, VCH)], cb)
            pltpu.sync_copy(cb, xvs.at[pl.ds(off, VCH)])

    @pl.loop(0, VCH, step=16)
    def _(i):
        cb[pl.ds(i, 16)] = jnp.zeros((16,), jnp.float32)

    for k in range((NVCH + NSUB - 1) // NSUB):
        t = sid + NSUB * k

        @pl.when(t < NVCH)
        def _():
            pltpu.sync_copy(cb, accn.at[pl.ds(t * VCH, VCH)])
            pltpu.sync_copy(cb, accd.at[pl.ds(t * VCH, VCH)])

    plsc.subcore_barrier()

    NCH2 = 2 * NCHUNK

    def _src(j):
        if j < NCHUNK:
            return adjp_hbm, True, j
        return adjn_hbm, False, j - NCHUNK

    descL = [None] * NCH2
    descG = [None] * NCH2
    descS = [None] * NCH2
    for j in range(NCH2 + 2):
        if j < NCH2:
            b = j % NBUF
            if j >= NBUF:
                descS[j - NBUF][0].wait()
                descS[j - NBUF][1].wait()
            adj, _, jj = _src(j)
            off = w * EPW + jj * ECH
            descL[j] = (
                pltpu.async_copy(adj.at[pl.ds(off, ECH)], cidx[b],
                                 semL.at[b]),
                pltpu.async_copy(adj.at[pl.ds(NE + off, ECH)], vidx[b],
                                 semL.at[b]),
            )
        if 0 <= j - 1 < NCH2:
            jc = j - 1
            b = jc % NBUF
            descL[jc][1].wait()
            descG[jc] = pltpu.async_copy(xvs.at[vidx[b]], xval[b],
                                         semG.at[b])
        if 0 <= j - 2 < NCH2:
            jc = j - 2
            b = jc % NBUF
            descG[jc].wait()
            _, pos, _ = _src(jc)
            xr, nar, nbr = xval[b], na[b], nb[b]

            @pl.loop(0, ECH, step=16)
            def _(i):
                x = xr[pl.ds(i, 16)]
                t = x if pos else 1.0 - x
                e = jnp.exp(PCOEF * t)
                nar[pl.ds(i, 16)] = t * e
                nbr[pl.ds(i, 16)] = e

            descL[jc][0].wait()
            descS[jc] = (
                pltpu.async_copy(na[b], accn.at[cidx[b]], semS.at[b],
                                 add=True),
                pltpu.async_copy(nb[b], accd.at[cidx[b]], semS.at[b],
                                 add=True),
            )
    for jc in range(NCH2 - NBUF, NCH2):
        descS[jc][0].wait()
        descS[jc][1].wait()

    plsc.subcore_barrier()

    NDCH = NC // ECH
    for k in range((2 * NDCH + NSUB - 1) // NSUB):
        t = sid + NSUB * k

        @pl.when(t < NDCH)
        def _():
            o = t * ECH
            pltpu.sync_copy(accn.at[pl.ds(o, ECH)], na[0])
            pltpu.sync_copy(na[0], out_hbm.at[pl.ds(2 * cid * NC + o, ECH)])

        @pl.when((t >= NDCH) & (t < 2 * NDCH))
        def _():
            o = (t - NDCH) * ECH
            pltpu.sync_copy(accd.at[pl.ds(o, ECH)], na[1])
            pltpu.sync_copy(na[1],
                            out_hbm.at[pl.ds((2 * cid + 1) * NC + o, ECH)])


def _sc_edge_phase(xvf, adj_pos, adj_neg):
    mesh = plsc.VectorSubcoreMesh(core_axis_name="c", subcore_axis_name="s")
    cp = pltpu.CompilerParams()
    if "needs_layout_passes" in pltpu.CompilerParams.__dataclass_fields__:
        cp = dataclasses.replace(cp, needs_layout_passes=False)
    return pl.kernel(
        _sc_body,
        out_type=jax.ShapeDtypeStruct((4 * NC,), jnp.float32),
        mesh=mesh,
        compiler_params=cp,
        scratch_types=(
            [pltpu.VMEM((ECH,), jnp.int32) for _ in range(2 * NBUF)] +
            [pltpu.VMEM((ECH,), jnp.float32) for _ in range(3 * NBUF)] +
            [pltpu.VMEM((VCH,), jnp.float32)] +
            [pltpu.VMEM_SHARED((NV,), jnp.float32),
             pltpu.VMEM_SHARED((NC,), jnp.float32),
             pltpu.VMEM_SHARED((NC,), jnp.float32),
             pltpu.SemaphoreType.DMA((NBUF,)),
             pltpu.SemaphoreType.DMA((NBUF,)),
             pltpu.SemaphoreType.DMA((NBUF,))]
        ),
    )(xvf, adj_pos, adj_neg)


_FR = 50
_FC = NC // _FR


def _tc_final_body(parts_ref, gidx_ref, cc_ref, out_ref):
    iota = lax.broadcasted_iota(jnp.int32, (NG, 1), 0)

    def step(k, carry):
        acc, pen = carry
        num = parts_ref[0, pl.ds(k, 1), :] + parts_ref[2, pl.ds(k, 1), :]
        dom = parts_ref[1, pl.ds(k, 1), :] + parts_ref[3, pl.ds(k, 1), :]
        sm = num / dom
        pen = pen + jnp.sum(jnp.maximum(10.0 * (sm - 0.45), 0.0))
        g = gidx_ref[pl.ds(k, 1), :]
        oh = (g == iota).astype(jnp.float32)
        acc = acc + lax.dot_general(sm, oh, (((1,), (1,)), ((), ())),
                                    preferred_element_type=jnp.float32)
        return acc, pen

    acc, pen_sum = lax.fori_loop(
        0, _FR, step, (jnp.zeros((1, NG), jnp.float32), jnp.float32(0.0)))
    pg = acc / cc_ref[...]
    loss = jnp.mean((pg - 1.0) ** 2)
    out_ref[...] = jnp.stack([loss, loss - pen_sum * 0.005]).reshape(1, 2)


def kernel(xv, adj_pos, adj_neg, clause_count, gr_idx_cls, is_train):
    xvf = xv.reshape(NV)
    sc_out = _sc_edge_phase(xvf, adj_pos.reshape(2 * NE),
                            adj_neg.reshape(2 * NE))
    parts = sc_out.reshape(4, _FR, _FC)
    gidx = gr_idx_cls.reshape(_FR, _FC)
    cc = clause_count.reshape(1, NG)
    r = pl.pallas_call(
        _tc_final_body,
        out_shape=jax.ShapeDtypeStruct((1, 2), jnp.float32),
    )(parts, gidx, cc)
    return jnp.where(is_train, r[0, 1], r[0, 0])


# speedup vs baseline: 1.1515x; 1.1515x over previous
import jax
import jax.numpy as jnp
from jax import lax
from jax.experimental import pallas as pl
from jax.experimental.pallas import tpu as pltpu
from jax.experimental.pallas import tpu_sc as plsc

NV = 100000
NC = 100000
NE = 3200000
NG = 256
PCOEF = 3.0

NSUB = 16
NW = 32
EPW = NE // NW
ECH = 4000
NCHUNK = EPW // ECH
VCH = 1000
NVCH = NV // VCH
NBUF = 5


def _sc_body(xv_hbm, adjp_hbm, adjn_hbm, out_hbm, *rest):
    cidx = list(rest[0:NBUF])
    vidx = list(rest[NBUF:2 * NBUF])
    na = list(rest[2 * NBUF:3 * NBUF])
    nb = list(rest[3 * NBUF:4 * NBUF])
    (xb, cb, ap, bp, an, bn, accn, accd, semL, semG, semS) = rest[4 * NBUF:]
    cid = lax.axis_index("c")
    sid = lax.axis_index("s")
    w = cid * NSUB + sid

    for k in range((NVCH + NSUB - 1) // NSUB):
        t = sid + NSUB * k

        @pl.when(t < NVCH)
        def _():
            off = t * VCH
            pltpu.sync_copy(xv_hbm.at[pl.ds(off, VCH)], xb)

            @pl.loop(0, VCH, step=16)
            def _(i):
                x = xb[pl.ds(i, 16)]
                cb[pl.ds(i, 16)] = x * jnp.exp(PCOEF * x)
            pltpu.sync_copy(cb, ap.at[pl.ds(off, VCH)])

            @pl.loop(0, VCH, step=16)
            def _(i):
                x = xb[pl.ds(i, 16)]
                cb[pl.ds(i, 16)] = jnp.exp(PCOEF * x)
            pltpu.sync_copy(cb, bp.at[pl.ds(off, VCH)])

            @pl.loop(0, VCH, step=16)
            def _(i):
                x = 1.0 - xb[pl.ds(i, 16)]
                cb[pl.ds(i, 16)] = x * jnp.exp(PCOEF * x)
            pltpu.sync_copy(cb, an.at[pl.ds(off, VCH)])

            @pl.loop(0, VCH, step=16)
            def _(i):
                x = 1.0 - xb[pl.ds(i, 16)]
                cb[pl.ds(i, 16)] = jnp.exp(PCOEF * x)
            pltpu.sync_copy(cb, bn.at[pl.ds(off, VCH)])

    @pl.loop(0, VCH, step=16)
    def _(i):
        cb[pl.ds(i, 16)] = jnp.zeros((16,), jnp.float32)

    for k in range((NVCH + NSUB - 1) // NSUB):
        t = sid + NSUB * k

        @pl.when(t < NVCH)
        def _():
            pltpu.sync_copy(cb, accn.at[pl.ds(t * VCH, VCH)])
            pltpu.sync_copy(cb, accd.at[pl.ds(t * VCH, VCH)])

    plsc.subcore_barrier()

    NCH2 = 2 * NCHUNK

    def _src(j):
        if j < NCHUNK:
            return adjp_hbm, ap, bp, j
        return adjn_hbm, an, bn, j - NCHUNK

    descL = [None] * NCH2
    descG = [None] * NCH2
    descS = [None] * NCH2
    for j in range(NCH2 + 2):
        if j < NCH2:
            b = j % NBUF
            if j >= NBUF:
                descS[j - NBUF][0].wait()
                descS[j - NBUF][1].wait()
            adj, _, _, jj = _src(j)
            off = w * EPW + jj * ECH
            descL[j] = (
                pltpu.async_copy(adj.at[pl.ds(off, ECH)], cidx[b],
                                 semL.at[b]),
                pltpu.async_copy(adj.at[pl.ds(NE + off, ECH)], vidx[b],
                                 semL.at[b]),
            )
        if 0 <= j - 1 < NCH2:
            jc = j - 1
            b = jc % NBUF
            descL[jc][0].wait()
            descL[jc][1].wait()
            _, ta, tb, _ = _src(jc)
            descG[jc] = (
                pltpu.async_copy(ta.at[vidx[b]], na[b], semG.at[b]),
                pltpu.async_copy(tb.at[vidx[b]], nb[b], semG.at[b]),
            )
        if 0 <= j - 2 < NCH2:
            jc = j - 2
            b = jc % NBUF
            descG[jc][0].wait()
            descG[jc][1].wait()
            descS[jc] = (
                pltpu.async_copy(na[b], accn.at[cidx[b]], semS.at[b],
                                 add=True),
                pltpu.async_copy(nb[b], accd.at[cidx[b]], semS.at[b],
                                 add=True),
            )
    for jc in range(NCH2 - NBUF, NCH2):
        descS[jc][0].wait()
        descS[jc][1].wait()

    plsc.subcore_barrier()

    NDCH = NC // ECH
    for k in range((2 * NDCH + NSUB - 1) // NSUB):
        t = sid + NSUB * k

        @pl.when(t < NDCH)
        def _():
            o = t * ECH
            pltpu.sync_copy(accn.at[pl.ds(o, ECH)], na[0])
            pltpu.sync_copy(na[0], out_hbm.at[pl.ds(2 * cid * NC + o, ECH)])

        @pl.when((t >= NDCH) & (t < 2 * NDCH))
        def _():
            o = (t - NDCH) * ECH
            pltpu.sync_copy(accd.at[pl.ds(o, ECH)], na[1 % NBUF])
            pltpu.sync_copy(na[1 % NBUF],
                            out_hbm.at[pl.ds((2 * cid + 1) * NC + o, ECH)])


def _sc_edge_phase(xvf, adj_pos, adj_neg):
    mesh = plsc.VectorSubcoreMesh(core_axis_name="c", subcore_axis_name="s")
    return pl.kernel(
        _sc_body,
        out_type=jax.ShapeDtypeStruct((4 * NC,), jnp.float32),
        mesh=mesh,
        scratch_types=(
            [pltpu.VMEM((ECH,), jnp.int32) for _ in range(2 * NBUF)] +
            [pltpu.VMEM((ECH,), jnp.float32) for _ in range(2 * NBUF)] +
            [pltpu.VMEM((VCH,), jnp.float32),
             pltpu.VMEM((VCH,), jnp.float32),
             pltpu.VMEM_SHARED((NV,), jnp.float32),
             pltpu.VMEM_SHARED((NV,), jnp.float32),
             pltpu.VMEM_SHARED((NV,), jnp.float32),
             pltpu.VMEM_SHARED((NV,), jnp.float32),
             pltpu.VMEM_SHARED((NC,), jnp.float32),
             pltpu.VMEM_SHARED((NC,), jnp.float32),
             pltpu.SemaphoreType.DMA((NBUF,)),
             pltpu.SemaphoreType.DMA((NBUF,)),
             pltpu.SemaphoreType.DMA((NBUF,))]
        ),
    )(xvf, adj_pos, adj_neg)


_FR = 50
_FC = NC // _FR


def _tc_final_body(parts_ref, gidx_ref, cc_ref, out_ref):
    iota = lax.broadcasted_iota(jnp.int32, (NG, 1), 0)

    def step(k, carry):
        acc, pen = carry
        num = parts_ref[0, pl.ds(k, 1), :] + parts_ref[2, pl.ds(k, 1), :]
        dom = parts_ref[1, pl.ds(k, 1), :] + parts_ref[3, pl.ds(k, 1), :]
        sm = num / dom
        pen = pen + jnp.sum(jnp.maximum(10.0 * (sm - 0.45), 0.0))
        g = gidx_ref[pl.ds(k, 1), :]
        oh = (g == iota).astype(jnp.float32)
        acc = acc + lax.dot_general(sm, oh, (((1,), (1,)), ((), ())),
                                    preferred_element_type=jnp.float32)
        return acc, pen

    acc, pen_sum = lax.fori_loop(
        0, _FR, step, (jnp.zeros((1, NG), jnp.float32), jnp.float32(0.0)))
    pg = acc / cc_ref[...]
    loss = jnp.mean((pg - 1.0) ** 2)
    out_ref[...] = jnp.stack([loss, loss - pen_sum * 0.005]).reshape(1, 2)


def kernel(xv, adj_pos, adj_neg, clause_count, gr_idx_cls, is_train):
    xvf = xv.reshape(NV)
    sc_out = _sc_edge_phase(xvf, adj_pos.reshape(2 * NE),
                            adj_neg.reshape(2 * NE))
    parts = sc_out.reshape(4, _FR, _FC)
    gidx = gr_idx_cls.reshape(_FR, _FC)
    cc = clause_count.reshape(1, NG)
    r = pl.pallas_call(
        _tc_final_body,
        out_shape=jax.ShapeDtypeStruct((1, 2), jnp.float32),
    )(parts, gidx, cc)
    return jnp.where(is_train, r[0, 1], r[0, 0])


# speedup vs baseline: 1.1858x; 1.0298x over previous
import jax
import jax.numpy as jnp
from jax import lax
from jax.experimental import pallas as pl
from jax.experimental.pallas import tpu as pltpu
from jax.experimental.pallas import tpu_sc as plsc

NV = 100000
NC = 100000
NE = 3200000
NG = 256
PCOEF = 3.0

NSUB = 16
NW = 32
EPW = NE // NW
ECH = 4000
NCHUNK = EPW // ECH
VCH = 1000
NVCH = NV // VCH
NBUF = 5


def _sc_body(xv_hbm, adjp_hbm, adjn_hbm, out_hbm, *rest):
    cidx = list(rest[0:NBUF])
    vidx = list(rest[NBUF:2 * NBUF])
    na = list(rest[2 * NBUF:3 * NBUF])
    nb = list(rest[3 * NBUF:4 * NBUF])
    (xb, cb, ap, bp, an, bn, accn, accd, semL, semG, semS) = rest[4 * NBUF:]
    cid = lax.axis_index("c")
    sid = lax.axis_index("s")
    w = cid * NSUB + sid

    for k in range((NVCH + NSUB - 1) // NSUB):
        t = sid + NSUB * k

        @pl.when(t < NVCH)
        def _():
            off = t * VCH
            pltpu.sync_copy(xv_hbm.at[pl.ds(off, VCH)], xb)

            @pl.loop(0, VCH, step=16)
            def _(i):
                x = xb[pl.ds(i, 16)]
                cb[pl.ds(i, 16)] = x * jnp.exp(PCOEF * x)
            pltpu.sync_copy(cb, ap.at[pl.ds(off, VCH)])

            @pl.loop(0, VCH, step=16)
            def _(i):
                x = xb[pl.ds(i, 16)]
                cb[pl.ds(i, 16)] = jnp.exp(PCOEF * x)
            pltpu.sync_copy(cb, bp.at[pl.ds(off, VCH)])

            @pl.loop(0, VCH, step=16)
            def _(i):
                x = 1.0 - xb[pl.ds(i, 16)]
                cb[pl.ds(i, 16)] = x * jnp.exp(PCOEF * x)
            pltpu.sync_copy(cb, an.at[pl.ds(off, VCH)])

            @pl.loop(0, VCH, step=16)
            def _(i):
                x = 1.0 - xb[pl.ds(i, 16)]
                cb[pl.ds(i, 16)] = jnp.exp(PCOEF * x)
            pltpu.sync_copy(cb, bn.at[pl.ds(off, VCH)])

    @pl.loop(0, VCH, step=16)
    def _(i):
        cb[pl.ds(i, 16)] = jnp.zeros((16,), jnp.float32)

    for k in range((NVCH + NSUB - 1) // NSUB):
        t = sid + NSUB * k

        @pl.when(t < NVCH)
        def _():
            pltpu.sync_copy(cb, accn.at[pl.ds(t * VCH, VCH)])
            pltpu.sync_copy(cb, accd.at[pl.ds(t * VCH, VCH)])

    plsc.subcore_barrier()

    NCH2 = 2 * NCHUNK

    def _src(j):
        if j < NCHUNK:
            return adjp_hbm, ap, bp, j
        return adjn_hbm, an, bn, j - NCHUNK

    descL = [None] * NCH2
    descG = [None] * NCH2
    descS = [None] * NCH2
    for j in range(NCH2 + 2):
        if j < NCH2:
            b = j % NBUF
            if j >= NBUF:
                descS[j - NBUF][0].wait()
                descS[j - NBUF][1].wait()
            adj, _, _, jj = _src(j)
            off = w * EPW + jj * ECH
            descL[j] = (
                pltpu.async_copy(adj.at[pl.ds(off, ECH)], cidx[b],
                                 semL.at[b]),
                pltpu.async_copy(adj.at[pl.ds(NE + off, ECH)], vidx[b],
                                 semL.at[b]),
            )
        if 0 <= j - 1 < NCH2:
            jc = j - 1
            b = jc % NBUF
            descL[jc][0].wait()
            descL[jc][1].wait()
            _, ta, tb, _ = _src(jc)
            descG[jc] = (
                pltpu.async_copy(ta.at[vidx[b]], na[b], semG.at[b]),
                pltpu.async_copy(tb.at[vidx[b]], nb[b], semG.at[b]),
            )
        if 0 <= j - 2 < NCH2:
            jc = j - 2
            b = jc % NBUF
            descG[jc][0].wait()
            descG[jc][1].wait()
            descS[jc] = (
                pltpu.async_copy(na[b], accn.at[cidx[b]], semS.at[b],
                                 add=True),
                pltpu.async_copy(nb[b], accd.at[cidx[b]], semS.at[b],
                                 add=True),
            )
    for jc in range(NCH2 - NBUF, NCH2):
        descS[jc][0].wait()
        descS[jc][1].wait()

    plsc.subcore_barrier()

    NDCH = NC // ECH
    for k in range((2 * NDCH + NSUB - 1) // NSUB):
        t = sid + NSUB * k

        @pl.when(t < NDCH)
        def _():
            o = t * ECH
            pltpu.sync_copy(accn.at[pl.ds(o, ECH)], na[0])
            pltpu.sync_copy(na[0], out_hbm.at[pl.ds(2 * cid * NC + o, ECH)])

        @pl.when((t >= NDCH) & (t < 2 * NDCH))
        def _():
            o = (t - NDCH) * ECH
            pltpu.sync_copy(accd.at[pl.ds(o, ECH)], na[1])
            pltpu.sync_copy(na[1],
                            out_hbm.at[pl.ds((2 * cid + 1) * NC + o, ECH)])


def _sc_edge_phase(xvf, adj_pos, adj_neg):
    mesh = plsc.VectorSubcoreMesh(core_axis_name="c", subcore_axis_name="s")
    return pl.kernel(
        _sc_body,
        out_type=jax.ShapeDtypeStruct((4 * NC,), jnp.float32),
        mesh=mesh,
        scratch_types=(
            [pltpu.VMEM((ECH,), jnp.int32) for _ in range(2 * NBUF)] +
            [pltpu.VMEM((ECH,), jnp.float32) for _ in range(2 * NBUF)] +
            [pltpu.VMEM((VCH,), jnp.float32),
             pltpu.VMEM((VCH,), jnp.float32),
             pltpu.VMEM_SHARED((NV,), jnp.float32),
             pltpu.VMEM_SHARED((NV,), jnp.float32),
             pltpu.VMEM_SHARED((NV,), jnp.float32),
             pltpu.VMEM_SHARED((NV,), jnp.float32),
             pltpu.VMEM_SHARED((NC,), jnp.float32),
             pltpu.VMEM_SHARED((NC,), jnp.float32),
             pltpu.SemaphoreType.DMA((NBUF,)),
             pltpu.SemaphoreType.DMA((NBUF,)),
             pltpu.SemaphoreType.DMA((NBUF,))]
        ),
    )(xvf, adj_pos, adj_neg)


_FR = 50
_FC = NC // _FR


def _tc_final_body(parts_ref, gidx_ref, cc_ref, out_ref):
    iota16 = lax.broadcasted_iota(jnp.int32, (16, 1), 0)

    def step(k, carry):
        acc, pen = carry
        num = parts_ref[0, pl.ds(k, 1), :] + parts_ref[2, pl.ds(k, 1), :]
        dom = parts_ref[1, pl.ds(k, 1), :] + parts_ref[3, pl.ds(k, 1), :]
        sm = num / dom
        pen = pen + jnp.sum(jnp.maximum(10.0 * (sm - 0.45), 0.0))
        g = gidx_ref[pl.ds(k, 1), :]
        mhi = jnp.where((g >> 4) == iota16, sm, 0.0)
        olo = ((g & 15) == iota16).astype(jnp.float32)
        acc = acc + lax.dot_general(mhi, olo, (((1,), (1,)), ((), ())),
                                    preferred_element_type=jnp.float32)
        return acc, pen

    acc, pen_sum = lax.fori_loop(
        0, _FR, step, (jnp.zeros((16, 16), jnp.float32), jnp.float32(0.0)))
    pg = acc / cc_ref[...]
    loss = jnp.mean((pg - 1.0) ** 2)
    out_ref[...] = jnp.stack([loss, loss - pen_sum * 0.005]).reshape(1, 2)


def kernel(xv, adj_pos, adj_neg, clause_count, gr_idx_cls, is_train):
    xvf = xv.reshape(NV)
    sc_out = _sc_edge_phase(xvf, adj_pos.reshape(2 * NE),
                            adj_neg.reshape(2 * NE))
    parts = sc_out.reshape(4, _FR, _FC)
    gidx = gr_idx_cls.reshape(_FR, _FC)
    cc = clause_count.reshape(16, 16)
    r = pl.pallas_call(
        _tc_final_body,
        out_shape=jax.ShapeDtypeStruct((1, 2), jnp.float32),
    )(parts, gidx, cc)
    return jnp.where(is_train, r[0, 1], r[0, 0])
